# Initial kernel scaffold; baseline (speedup 1.0000x reference)
import functools

import jax
import jax.numpy as jnp
from jax import lax
from jax.experimental import pallas as pl
from jax.experimental.pallas import tpu as pltpu
from jax.experimental.pallas import tpu_sc as plsc

N_NODES = 10000
N_EDGES = 320000
EDIM = 16
NDIM_IN = 128
NDIM_OUT = 128
ACCW = 32

NWORK = 32
EPT = N_EDGES // NWORK
SUB = 80
CH = 2000
NCH = EPT // CH
RPC = CH // SUB
ROWS_PER_TILE = N_NODES // 16


def _msg_body(ef_ref, nw_ref, out_ref):
    ef = ef_ref[...]
    nw = nw_ref[...]
    scaled = ef * nw
    cols = lax.broadcasted_iota(jnp.int32, (ef.shape[0], EDIM), 1)
    onehot = jnp.where(cols == 0, 1.0, 0.0).astype(jnp.float32)
    out_ref[...] = jnp.concatenate([scaled, onehot], axis=1)


def _make_msg(ef2, nw2):
    blk = 20000
    grid = N_EDGES // blk
    return pl.pallas_call(
        _msg_body,
        grid=(grid,),
        in_specs=[
            pl.BlockSpec((blk, EDIM), lambda i: (i, 0)),
            pl.BlockSpec((blk, 1), lambda i: (i, 0)),
        ],
        out_specs=pl.BlockSpec((blk, ACCW), lambda i: (i, 0)),
        out_shape=jax.ShapeDtypeStruct((N_EDGES, ACCW), jnp.float32),
    )(ef2, nw2)


def _sc_scatter_body(msg_hbm, dst_hbm, zeros_hbm, out_hbm, msg_v, dst_v, acc_sh):
    cid = lax.axis_index("c")
    sid = lax.axis_index("s")
    wid = cid * 16 + sid
    zbase = sid * ROWS_PER_TILE
    pltpu.sync_copy(zeros_hbm.at[pl.ds(zbase, ROWS_PER_TILE)],
                    acc_sh.at[pl.ds(zbase, ROWS_PER_TILE)])
    plsc.subcore_barrier()

    def chunk_body(ci, carry):
        base = wid * EPT + ci * CH
        row0 = wid * (EPT // SUB) + ci * RPC
        pltpu.sync_copy(msg_hbm.at[pl.ds(base, CH)], msg_v)
        pltpu.sync_copy(dst_hbm.at[pl.ds(row0, RPC)], dst_v)
        for j in range(RPC):
            pltpu.sync_copy(msg_v.at[pl.ds(j * SUB, SUB)],
                            acc_sh.at[dst_v.at[j]], add=True)
        return carry

    lax.fori_loop(0, NCH, chunk_body, 0)
    plsc.subcore_barrier()
    pltpu.sync_copy(acc_sh.at[pl.ds(zbase, ROWS_PER_TILE)],
                    out_hbm.at[cid, pl.ds(zbase, ROWS_PER_TILE)])


_sc_scatter = functools.partial(
    pl.kernel,
    out_type=jax.ShapeDtypeStruct((2, N_NODES, ACCW), jnp.float32),
    mesh=plsc.VectorSubcoreMesh(core_axis_name="c", subcore_axis_name="s"),
    scratch_types=[
        pltpu.VMEM((CH, ACCW), jnp.float32),
        pltpu.VMEM((RPC, SUB), jnp.int32),
        pltpu.VMEM_SHARED((N_NODES, ACCW), jnp.float32),
    ],
)(_sc_scatter_body)


def _final_body(parts_ref, nf_ref, wt_ref, b_ref, out_ref):
    s = parts_ref[0] + parts_ref[1]
    deg = jnp.maximum(s[:, EDIM:EDIM + 1], 1.0)
    h_neigh = s[:, :EDIM] / deg
    h = jnp.concatenate([nf_ref[...], h_neigh], axis=1)
    acc = jnp.dot(h, wt_ref[...], preferred_element_type=jnp.float32)
    out_ref[...] = jnp.maximum(acc + b_ref[...], 0.0)


def _final(parts, nf2, wt, b2):
    return pl.pallas_call(
        _final_body,
        out_shape=jax.ShapeDtypeStruct((N_NODES, NDIM_OUT), jnp.float32),
    )(parts, nf2, wt, b2)


def kernel(nfeats, efeats, edge_index, norm_weight, W, b):
    ef2 = efeats.reshape(N_EDGES, EDIM)
    nw2 = norm_weight.reshape(N_EDGES, 1)
    dst = edge_index[1].astype(jnp.int32).reshape(N_EDGES // SUB, SUB)
    zeros = jnp.zeros((N_NODES, ACCW), jnp.float32)
    wt = W.T
    b2 = b.reshape(1, NDIM_OUT)

    msg = _make_msg(ef2, nw2)
    parts = _sc_scatter(msg, dst, zeros)
    out2 = _final(parts, nfeats.reshape(N_NODES, NDIM_IN), wt, b2)
    return out2.reshape(N_NODES, 1, NDIM_OUT)


# speedup vs baseline: 2.5174x; 2.5174x over previous
import functools

import jax
import jax.numpy as jnp
from jax import lax
from jax.experimental import pallas as pl
from jax.experimental.pallas import tpu as pltpu
from jax.experimental.pallas import tpu_sc as plsc

N_NODES = 10000
N_EDGES = 320000
EDIM = 16
NDIM_IN = 128
NDIM_OUT = 128
ACCW = 32

NWORK = 32
EPT = N_EDGES // NWORK
SUB = 80
CH = 2000
NCH = EPT // CH
RPC = CH // SUB
N_PAD = 10240
ROWS_PER_TILE = N_PAD // 16


def _msg_body(ef_ref, nw_ref, out_ref):
    ef = ef_ref[...]
    nw = nw_ref[...]
    scaled = ef * nw
    cols = lax.broadcasted_iota(jnp.int32, (ef.shape[0], EDIM), 1)
    onehot = jnp.where(cols == 0, 1.0, 0.0).astype(jnp.float32)
    out_ref[...] = jnp.concatenate([scaled, onehot], axis=1)


def _make_msg(ef2, nw2):
    blk = 2000
    grid = N_EDGES // blk
    return pl.pallas_call(
        _msg_body,
        grid=(grid,),
        in_specs=[
            pl.BlockSpec((blk, EDIM), lambda i: (i, 0)),
            pl.BlockSpec((blk, 1), lambda i: (i, 0)),
        ],
        out_specs=pl.BlockSpec((blk, ACCW), lambda i: (i, 0)),
        out_shape=jax.ShapeDtypeStruct((N_EDGES, ACCW), jnp.float32),
    )(ef2, nw2)


def _sc_scatter_body(msg_hbm, dst_hbm, zeros_hbm, out_hbm, msg_v, dst_v, acc_sh):
    cid = lax.axis_index("c")
    sid = lax.axis_index("s")
    wid = cid * 16 + sid
    zbase = sid * ROWS_PER_TILE
    pltpu.sync_copy(zeros_hbm.at[pl.ds(zbase, ROWS_PER_TILE)],
                    acc_sh.at[pl.ds(zbase, ROWS_PER_TILE)])
    plsc.subcore_barrier()

    pltpu.sync_copy(dst_hbm.at[wid], dst_v)
    for ci in range(NCH):
        base = wid * EPT + ci * CH
        pltpu.sync_copy(msg_hbm.at[pl.ds(base, CH)], msg_v)
        for j in range(RPC):
            pltpu.sync_copy(msg_v.at[pl.ds(j * SUB, SUB)],
                            acc_sh.at[dst_v.at[ci * RPC + j]], add=True)
    plsc.subcore_barrier()
    pltpu.sync_copy(acc_sh.at[pl.ds(zbase, ROWS_PER_TILE)],
                    out_hbm.at[cid, pl.ds(zbase, ROWS_PER_TILE)])


_sc_scatter = functools.partial(
    pl.kernel,
    out_type=jax.ShapeDtypeStruct((2, N_PAD, ACCW), jnp.float32),
    mesh=plsc.VectorSubcoreMesh(core_axis_name="c", subcore_axis_name="s"),
    compiler_params=pltpu.CompilerParams(use_tc_tiling_on_sc=False),
    scratch_types=[
        pltpu.VMEM((CH, ACCW), jnp.float32),
        pltpu.VMEM((EPT // SUB, SUB), jnp.int32),
        pltpu.VMEM_SHARED((N_PAD, ACCW), jnp.float32),
    ],
)(_sc_scatter_body)


def _final_body(parts_ref, nf_ref, wt_ref, b_ref, out_ref):
    s = parts_ref[0] + parts_ref[1]
    deg = jnp.maximum(s[:, EDIM:EDIM + 1], 1.0)
    h_neigh = s[:, :EDIM] / deg
    h = jnp.concatenate([nf_ref[...], h_neigh], axis=1)
    acc = jnp.dot(h, wt_ref[...], preferred_element_type=jnp.float32)
    out_ref[...] = jnp.maximum(acc + b_ref[...], 0.0)


def _final(parts, nf2, wt, b2):
    return pl.pallas_call(
        _final_body,
        out_shape=jax.ShapeDtypeStruct((N_NODES, NDIM_OUT), jnp.float32),
    )(parts, nf2, wt, b2)


def kernel(nfeats, efeats, edge_index, norm_weight, W, b):
    ef2 = efeats.reshape(N_EDGES, EDIM)
    nw2 = norm_weight.reshape(N_EDGES, 1)
    dst = edge_index[1].astype(jnp.int32).reshape(NWORK, EPT // SUB, SUB)
    zeros = jnp.zeros((N_PAD, ACCW), jnp.float32)
    wt = W.T
    b2 = b.reshape(1, NDIM_OUT)

    msg = _make_msg(ef2, nw2)
    parts = _sc_scatter(msg, dst, zeros)
    out2 = _final(parts[:, :N_NODES], nfeats.reshape(N_NODES, NDIM_IN), wt, b2)
    return out2.reshape(N_NODES, 1, NDIM_OUT)


# speedup vs baseline: 4.9920x; 1.9830x over previous
import functools

import jax
import jax.numpy as jnp
from jax import lax
from jax.experimental import pallas as pl
from jax.experimental.pallas import tpu as pltpu
from jax.experimental.pallas import tpu_sc as plsc

N_NODES = 10000
N_EDGES = 320000
EDIM = 16
NDIM_IN = 128
NDIM_OUT = 128

NWORK = 32
EPT = N_EDGES // NWORK
SUB = 80
CH = 2000
NCH = EPT // CH
RPC = CH // SUB
IPT = EPT // SUB
N_PAD = 10240
ROWS_PER_TILE = N_PAD // 16


def _sc_scatter_body(ef_hbm, nw_hbm, dst_hbm, zeros_hbm, ones_hbm,
                     out_hbm, dout_hbm,
                     ef_v, nw_v, dst_v, ones_v, acc_sh, dacc_sh):
    cid = lax.axis_index("c")
    sid = lax.axis_index("s")
    wid = cid * 16 + sid

    zbase = sid * ROWS_PER_TILE
    pltpu.sync_copy(zeros_hbm.at[pl.ds(zbase, ROWS_PER_TILE)],
                    acc_sh.at[pl.ds(zbase, ROWS_PER_TILE)])
    pltpu.sync_copy(zeros_hbm.at[pl.ds(zbase, ROWS_PER_TILE)],
                    dacc_sh.at[pl.ds(zbase, ROWS_PER_TILE)])
    pltpu.sync_copy(ones_hbm, ones_v)
    pltpu.sync_copy(dst_hbm.at[wid], dst_v)
    pltpu.sync_copy(nw_hbm.at[pl.ds(wid * EPT, EPT)], nw_v)
    plsc.subcore_barrier()

    for ci in range(NCH):
        base = wid * EPT + ci * CH
        pltpu.sync_copy(ef_hbm.at[pl.ds(base, CH)], ef_v)

        def row_body(e, carry, _ci=ci):
            nwv = plsc.load_gather(nw_v, [jnp.full((16,), _ci * CH + e,
                                                   dtype=jnp.int32)])
            ef_v[e, :] = ef_v[e, :] * nwv
            return carry

        lax.fori_loop(0, CH, row_body, 0)
        for j in range(RPC):
            pltpu.sync_copy(ef_v.at[pl.ds(j * SUB, SUB)],
                            acc_sh.at[dst_v.at[ci * RPC + j]], add=True)
            pltpu.sync_copy(ones_v,
                            dacc_sh.at[dst_v.at[ci * RPC + j]], add=True)

    plsc.subcore_barrier()
    pltpu.sync_copy(acc_sh.at[pl.ds(zbase, ROWS_PER_TILE)],
                    out_hbm.at[cid, pl.ds(zbase, ROWS_PER_TILE)])
    pltpu.sync_copy(dacc_sh.at[pl.ds(zbase, ROWS_PER_TILE)],
                    dout_hbm.at[cid, pl.ds(zbase, ROWS_PER_TILE)])


_sc_scatter = functools.partial(
    pl.kernel,
    out_type=(jax.ShapeDtypeStruct((2, N_PAD, EDIM), jnp.float32),
              jax.ShapeDtypeStruct((2, N_PAD, EDIM), jnp.float32)),
    mesh=plsc.VectorSubcoreMesh(core_axis_name="c", subcore_axis_name="s"),
    compiler_params=pltpu.CompilerParams(use_tc_tiling_on_sc=False,
                                         needs_layout_passes=False),
    scratch_types=[
        pltpu.VMEM((CH, EDIM), jnp.float32),
        pltpu.VMEM((EPT,), jnp.float32),
        pltpu.VMEM((IPT, SUB), jnp.int32),
        pltpu.VMEM((SUB, EDIM), jnp.float32),
        pltpu.VMEM_SHARED((N_PAD, EDIM), jnp.float32),
        pltpu.VMEM_SHARED((N_PAD, EDIM), jnp.float32),
    ],
)(_sc_scatter_body)


def _final_body(parts_ref, degp_ref, nf_ref, wt_ref, b_ref, out_ref):
    s = parts_ref[0] + parts_ref[1]
    deg = degp_ref[0][:, 0:1] + degp_ref[1][:, 0:1]
    h_neigh = s[:N_NODES] / jnp.maximum(deg[:N_NODES], 1.0)
    h = jnp.concatenate([nf_ref[...], h_neigh], axis=1)
    acc = jnp.dot(h, wt_ref[...], preferred_element_type=jnp.float32)
    out_ref[...] = jnp.maximum(acc + b_ref[...], 0.0)


def _final(parts, degp, nf2, wt, b2):
    return pl.pallas_call(
        _final_body,
        out_shape=jax.ShapeDtypeStruct((N_NODES, NDIM_OUT), jnp.float32),
    )(parts, degp, nf2, wt, b2)


def kernel(nfeats, efeats, edge_index, norm_weight, W, b):
    ef2 = efeats.reshape(N_EDGES, EDIM)
    dst = edge_index[1].astype(jnp.int32).reshape(NWORK, IPT, SUB)
    zeros = jnp.zeros((N_PAD, EDIM), jnp.float32)
    onehot = jnp.tile(jnp.eye(1, EDIM, dtype=jnp.float32), (SUB, 1))
    wt = W.T
    b2 = b.reshape(1, NDIM_OUT)

    parts, degp = _sc_scatter(ef2, norm_weight, dst, zeros, onehot)
    out2 = _final(parts, degp, nfeats.reshape(N_NODES, NDIM_IN), wt, b2)
    return out2.reshape(N_NODES, 1, NDIM_OUT)
